# baseline (speedup 1.0000x reference)
import dataclasses
import functools

import jax
import jax.numpy as jnp
from jax import lax
from jax.experimental import pallas as pl
from jax.experimental.pallas import tpu as pltpu
from jax.experimental.pallas import tpu_sc as plsc

_NC = 2
_NS = 16
_L = 16
_B = 128


@functools.partial(jax.jit, static_argnames=("E",))
def _fuser_sc(lt, rt, li, ri, E):
    NW = _NC * _NS
    T = E // _B
    assert T * _B == E

    mesh = plsc.VectorSubcoreMesh(core_axis_name="c", subcore_axis_name="s")
    cp = pltpu.CompilerParams()
    for fld, val in (("needs_layout_passes", False), ("use_tc_tiling_on_sc", True)):
        if fld in pltpu.CompilerParams.__dataclass_fields__:
            cp = dataclasses.replace(cp, **{fld: val})

    @functools.partial(
        pl.kernel,
        mesh=mesh,
        compiler_params=cp,
        out_type=(
            jax.ShapeDtypeStruct((32, E), jnp.float32),
            jax.ShapeDtypeStruct((3, 32, E), jnp.float32),
        ),
        scratch_types=[
            pltpu.VMEM((_B,), jnp.int32),
            pltpu.VMEM((_B,), jnp.int32),
            pltpu.VMEM((_B, 128), jnp.float32),
            pltpu.VMEM((_B, 128), jnp.float32),
            pltpu.VMEM((64, _B), jnp.float32),
            pltpu.VMEM((64, _B), jnp.float32),
            pltpu.VMEM((32, _B), jnp.float32),
            pltpu.VMEM((3, 32, _B), jnp.float32),
            pltpu.SemaphoreType.DMA,
            pltpu.SemaphoreType.DMA,
            pltpu.SemaphoreType.DMA,
            pltpu.SemaphoreType.DMA,
        ],
    )
    def k(lt_hbm, rt_hbm, li_hbm, ri_hbm, os_hbm, ov_hbm,
          lidx, ridx, lbuf, rbuf, lbufT, rbufT, sobuf, vobuf, sem0, sem1, sem2, sem3):
        wid = lax.axis_index("s") * _NC + lax.axis_index("c")
        iota = lax.iota(jnp.int32, _L)
        half = jnp.float32(0.5)
        gw = T // NW + jnp.where(wid < T % NW, 1, 0)

        @pl.loop(0, gw)
        def _chunk(g):
            base = (wid + g * NW) * _B
            cl = pltpu.async_copy(li_hbm.at[pl.ds(base, _B)], lidx, sem0)
            cr = pltpu.async_copy(ri_hbm.at[pl.ds(base, _B)], ridx, sem1)
            cl.wait()
            cr.wait()
            gl = pltpu.async_copy(lt_hbm.at[lidx], lbuf, sem0)
            gr_ = pltpu.async_copy(rt_hbm.at[ridx], rbuf, sem1)
            @pl.when(g > 0)
            def _():
                pltpu.make_async_copy(sobuf, os_hbm.at[:, pl.ds(base, _B)], sem2).wait()
                pltpu.make_async_copy(vobuf, ov_hbm.at[:, :, pl.ds(base, _B)], sem3).wait()
            gl.wait()
            gr_.wait()

            @plsc.parallel_loop(0, _B // _L)
            def _tr(gr):
                ev = gr * _L + iota
                for q in range(4):
                    for kk in range(16):
                        col = q * _L + ((iota + kk) & 15)
                        xl = plsc.load_gather(lbuf, [ev, col])
                        plsc.store_scatter(lbufT, [col, ev], xl)
                        xr = plsc.load_gather(rbuf, [ev, col])
                        plsc.store_scatter(rbufT, [col, ev], xr)

            @plsc.parallel_loop(0, _B // _L)
            def _grp(gr):
                sl = pl.ds(gr * _L, _L)
                for c in range(16):
                    ls = lbufT[c, sl]
                    lx = lbufT[16 + c, sl]
                    ly = lbufT[32 + c, sl]
                    lz = lbufT[48 + c, sl]
                    rs = rbufT[c, sl]
                    rx = rbufT[16 + c, sl]
                    ry = rbufT[32 + c, sl]
                    rz = rbufT[48 + c, sl]
                    sobuf[c, sl] = ls * rs
                    sobuf[16 + c, sl] = lx * rx + ly * ry + lz * rz
                    vobuf[0, c, sl] = (lx + rx) * half
                    vobuf[1, c, sl] = (ly + ry) * half
                    vobuf[2, c, sl] = (lz + rz) * half
                    vobuf[0, 16 + c, sl] = ly * rz - lz * ry
                    vobuf[1, 16 + c, sl] = lz * rx - lx * rz
                    vobuf[2, 16 + c, sl] = lx * ry - ly * rx

            pltpu.async_copy(sobuf, os_hbm.at[:, pl.ds(base, _B)], sem2)
            pltpu.async_copy(vobuf, ov_hbm.at[:, :, pl.ds(base, _B)], sem3)

        @pl.when(gw > 0)
        def _():
            last = (wid + (gw - 1) * NW) * _B
            pltpu.make_async_copy(sobuf, os_hbm.at[:, pl.ds(last, _B)], sem2).wait()
            pltpu.make_async_copy(vobuf, ov_hbm.at[:, :, pl.ds(last, _B)], sem3).wait()

    return k(lt, rt, li, ri)


def kernel(left_scalar, left_vector, right_scalar, right_vector, left_index, right_index):
    N, S = left_scalar.shape
    V = left_vector.shape[1]
    E = left_index.shape[0]
    pad = jnp.zeros((N, 128 - S - 3 * V), jnp.float32)
    lt = jnp.concatenate(
        [left_scalar, jnp.swapaxes(left_vector, 1, 2).reshape(N, 3 * V), pad], axis=1)
    rt = jnp.concatenate(
        [right_scalar, jnp.swapaxes(right_vector, 1, 2).reshape(N, 3 * V), pad], axis=1)
    so, vo = _fuser_sc(lt, rt, left_index, right_index, E)
    return (so.T, jnp.transpose(vo, (2, 1, 0)))


# speedup vs baseline: 1.2067x; 1.2067x over previous
import dataclasses
import functools

import jax
import jax.numpy as jnp
from jax import lax
from jax.experimental import pallas as pl
from jax.experimental.pallas import tpu as pltpu
from jax.experimental.pallas import tpu_sc as plsc

_NC = 2
_NS = 16
_L = 16
_B = 128


@functools.partial(jax.jit, static_argnames=("E",))
def _fuser_sc(lt, rt, li, ri, E):
    NW = _NC * _NS
    T = E // _B
    assert T * _B == E

    mesh = plsc.VectorSubcoreMesh(core_axis_name="c", subcore_axis_name="s")
    cp = pltpu.CompilerParams()
    for fld, val in (("needs_layout_passes", False), ("use_tc_tiling_on_sc", True)):
        if fld in pltpu.CompilerParams.__dataclass_fields__:
            cp = dataclasses.replace(cp, **{fld: val})

    sc2 = lambda shape, dt: [pltpu.VMEM(shape, dt), pltpu.VMEM(shape, dt)]

    @functools.partial(
        pl.kernel,
        mesh=mesh,
        compiler_params=cp,
        out_type=(
            jax.ShapeDtypeStruct((32, E), jnp.float32),
            jax.ShapeDtypeStruct((3, 32, E), jnp.float32),
        ),
        scratch_types=(
            sc2((_B,), jnp.int32) + sc2((_B,), jnp.int32)
            + sc2((_B, 128), jnp.float32) + sc2((_B, 128), jnp.float32)
            + [pltpu.VMEM((64, _B), jnp.float32), pltpu.VMEM((64, _B), jnp.float32)]
            + sc2((32, _B), jnp.float32) + sc2((3, 32, _B), jnp.float32)
            + [pltpu.SemaphoreType.DMA] * 12
        ),
    )
    def k(lt_hbm, rt_hbm, li_hbm, ri_hbm, os_hbm, ov_hbm,
          lidx0, lidx1, ridx0, ridx1, lbuf0, lbuf1, rbuf0, rbuf1,
          lbufT, rbufT, sobuf0, sobuf1, vobuf0, vobuf1,
          sil0, sil1, sir0, sir1, sgl0, sgl1, sgr0, sgr1,
          sos0, sos1, sov0, sov1):
        lidx = (lidx0, lidx1)
        ridx = (ridx0, ridx1)
        lbuf = (lbuf0, lbuf1)
        rbuf = (rbuf0, rbuf1)
        sobuf = (sobuf0, sobuf1)
        vobuf = (vobuf0, vobuf1)
        sil = (sil0, sil1)
        sir = (sir0, sir1)
        sgl = (sgl0, sgl1)
        sgr = (sgr0, sgr1)
        sos = (sos0, sos1)
        sov = (sov0, sov1)

        wid = lax.axis_index("s") * _NC + lax.axis_index("c")
        iota = lax.iota(jnp.int32, _L)
        half = jnp.float32(0.5)
        gw = T // NW + jnp.where(wid < T % NW, 1, 0)

        def cbase(gi):
            return (wid + gi * NW) * _B

        def issue_idx(gi, b):
            pltpu.async_copy(li_hbm.at[pl.ds(cbase(gi), _B)], lidx[b], sil[b])
            pltpu.async_copy(ri_hbm.at[pl.ds(cbase(gi), _B)], ridx[b], sir[b])

        def wait_idx(gi, b):
            pltpu.make_async_copy(li_hbm.at[pl.ds(cbase(gi), _B)], lidx[b], sil[b]).wait()
            pltpu.make_async_copy(ri_hbm.at[pl.ds(cbase(gi), _B)], ridx[b], sir[b]).wait()

        def issue_gather(b):
            pltpu.async_copy(lt_hbm.at[lidx[b]], lbuf[b], sgl[b])
            pltpu.async_copy(rt_hbm.at[ridx[b]], rbuf[b], sgr[b])

        def wait_gather(b):
            pltpu.make_async_copy(lt_hbm.at[lidx[b]], lbuf[b], sgl[b]).wait()
            pltpu.make_async_copy(rt_hbm.at[ridx[b]], rbuf[b], sgr[b]).wait()

        def issue_out(gi, b):
            pltpu.async_copy(sobuf[b], os_hbm.at[:, pl.ds(cbase(gi), _B)], sos[b])
            pltpu.async_copy(vobuf[b], ov_hbm.at[:, :, pl.ds(cbase(gi), _B)], sov[b])

        def wait_out(gi, b):
            pltpu.make_async_copy(sobuf[b], os_hbm.at[:, pl.ds(cbase(gi), _B)], sos[b]).wait()
            pltpu.make_async_copy(vobuf[b], ov_hbm.at[:, :, pl.ds(cbase(gi), _B)], sov[b]).wait()

        def work(b):
            @plsc.parallel_loop(0, _B // _L)
            def _tr(gr):
                ev = gr * _L + iota
                for q in range(4):
                    for kk in range(16):
                        col = q * _L + ((iota + kk) & 15)
                        xl = plsc.load_gather(lbuf[b], [ev, col])
                        plsc.store_scatter(lbufT, [col, ev], xl)
                        xr = plsc.load_gather(rbuf[b], [ev, col])
                        plsc.store_scatter(rbufT, [col, ev], xr)

            @plsc.parallel_loop(0, _B // _L)
            def _grp(gr):
                sl = pl.ds(gr * _L, _L)
                for c in range(16):
                    ls = lbufT[c, sl]
                    lx = lbufT[16 + c, sl]
                    ly = lbufT[32 + c, sl]
                    lz = lbufT[48 + c, sl]
                    rs = rbufT[c, sl]
                    rx = rbufT[16 + c, sl]
                    ry = rbufT[32 + c, sl]
                    rz = rbufT[48 + c, sl]
                    sobuf[b][c, sl] = ls * rs
                    sobuf[b][16 + c, sl] = lx * rx + ly * ry + lz * rz
                    vobuf[b][0, c, sl] = (lx + rx) * half
                    vobuf[b][1, c, sl] = (ly + ry) * half
                    vobuf[b][2, c, sl] = (lz + rz) * half
                    vobuf[b][0, 16 + c, sl] = ly * rz - lz * ry
                    vobuf[b][1, 16 + c, sl] = lz * rx - lx * rz
                    vobuf[b][2, 16 + c, sl] = lx * ry - ly * rx

        issue_idx(0, 0)

        @pl.when(gw > 1)
        def _():
            issue_idx(1, 1)
        wait_idx(0, 0)
        issue_gather(0)

        @pl.loop(0, (gw + 1) // 2)
        def _pair(g2):
            for b in (0, 1):
                gi = g2 * 2 + b

                @pl.when(gi < gw)
                def _():
                    @pl.when(gi + 1 < gw)
                    def _():
                        wait_idx(gi + 1, 1 - b)
                        issue_gather(1 - b)
                    wait_gather(b)

                    @pl.when(gi + 2 < gw)
                    def _():
                        issue_idx(gi + 2, b)

                    @pl.when(gi >= 2)
                    def _():
                        wait_out(gi - 2, b)
                    work(b)
                    issue_out(gi, b)

        for b in (0, 1):
            @pl.when((gw >= 2) & ((gw - 2) % 2 == b))
            def _():
                wait_out(gw - 2, b)

            @pl.when((gw >= 1) & ((gw - 1) % 2 == b))
            def _():
                wait_out(gw - 1, b)

    return k(lt, rt, li, ri)


def kernel(left_scalar, left_vector, right_scalar, right_vector, left_index, right_index):
    N, S = left_scalar.shape
    V = left_vector.shape[1]
    E = left_index.shape[0]
    pad = jnp.zeros((N, 128 - S - 3 * V), jnp.float32)
    lt = jnp.concatenate(
        [left_scalar, jnp.swapaxes(left_vector, 1, 2).reshape(N, 3 * V), pad], axis=1)
    rt = jnp.concatenate(
        [right_scalar, jnp.swapaxes(right_vector, 1, 2).reshape(N, 3 * V), pad], axis=1)
    so, vo = _fuser_sc(lt, rt, left_index, right_index, E)
    return (so.T, jnp.transpose(vo, (2, 1, 0)))


# speedup vs baseline: 1.8113x; 1.5010x over previous
import dataclasses
import functools

import jax
import jax.numpy as jnp
from jax import lax
from jax.experimental import pallas as pl
from jax.experimental.pallas import tpu as pltpu
from jax.experimental.pallas import tpu_sc as plsc

_NC = 2
_NS = 16
_L = 16
_B = 128


@functools.partial(jax.jit, static_argnames=("E",))
def _fuser_sc(lt, rt, li, ri, E):
    NW = _NC * _NS
    T = E // _B
    assert T * _B == E

    mesh = plsc.VectorSubcoreMesh(core_axis_name="c", subcore_axis_name="s")
    cp = pltpu.CompilerParams()
    for fld, val in (("needs_layout_passes", False), ("use_tc_tiling_on_sc", True)):
        if fld in pltpu.CompilerParams.__dataclass_fields__:
            cp = dataclasses.replace(cp, **{fld: val})

    sc2 = lambda shape, dt: [pltpu.VMEM(shape, dt), pltpu.VMEM(shape, dt)]

    @functools.partial(
        pl.kernel,
        mesh=mesh,
        compiler_params=cp,
        out_type=(
            jax.ShapeDtypeStruct((32, E), jnp.float32),
            jax.ShapeDtypeStruct((3, 32, E), jnp.float32),
        ),
        scratch_types=(
            sc2((_B,), jnp.int32) + sc2((_B,), jnp.int32)
            + sc2((_B, 128), jnp.float32) + sc2((_B, 128), jnp.float32)
            + [pltpu.VMEM((64, _B), jnp.float32), pltpu.VMEM((64, _B), jnp.float32)]
            + sc2((32, _B), jnp.float32) + sc2((3, 32, _B), jnp.float32)
            + [pltpu.SemaphoreType.DMA] * 12
        ),
    )
    def k(lt_hbm, rt_hbm, li_hbm, ri_hbm, os_hbm, ov_hbm,
          lidx0, lidx1, ridx0, ridx1, lbuf0, lbuf1, rbuf0, rbuf1,
          lbufT, rbufT, sobuf0, sobuf1, vobuf0, vobuf1,
          sil0, sil1, sir0, sir1, sgl0, sgl1, sgr0, sgr1,
          sos0, sos1, sov0, sov1):
        lidx = (lidx0, lidx1)
        ridx = (ridx0, ridx1)
        lbuf = (lbuf0, lbuf1)
        rbuf = (rbuf0, rbuf1)
        sobuf = (sobuf0, sobuf1)
        vobuf = (vobuf0, vobuf1)
        sil = (sil0, sil1)
        sir = (sir0, sir1)
        sgl = (sgl0, sgl1)
        sgr = (sgr0, sgr1)
        sos = (sos0, sos1)
        sov = (sov0, sov1)

        wid = lax.axis_index("s") * _NC + lax.axis_index("c")
        iota = lax.iota(jnp.int32, _L)
        half = jnp.float32(0.5)
        gw = T // NW + jnp.where(wid < T % NW, 1, 0)

        def cbase(gi):
            return (wid + gi * NW) * _B

        def issue_idx(gi, b):
            pltpu.async_copy(li_hbm.at[pl.ds(cbase(gi), _B)], lidx[b], sil[b])
            pltpu.async_copy(ri_hbm.at[pl.ds(cbase(gi), _B)], ridx[b], sir[b])

        def wait_idx(gi, b):
            pltpu.make_async_copy(li_hbm.at[pl.ds(cbase(gi), _B)], lidx[b], sil[b]).wait()
            pltpu.make_async_copy(ri_hbm.at[pl.ds(cbase(gi), _B)], ridx[b], sir[b]).wait()

        def issue_gather(b):
            pltpu.async_copy(lt_hbm.at[lidx[b]], lbuf[b], sgl[b])
            pltpu.async_copy(rt_hbm.at[ridx[b]], rbuf[b], sgr[b])

        def wait_gather(b):
            pltpu.make_async_copy(lt_hbm.at[lidx[b]], lbuf[b], sgl[b]).wait()
            pltpu.make_async_copy(rt_hbm.at[ridx[b]], rbuf[b], sgr[b]).wait()

        def issue_out(gi, b):
            pltpu.async_copy(sobuf[b], os_hbm.at[:, pl.ds(cbase(gi), _B)], sos[b])
            pltpu.async_copy(vobuf[b], ov_hbm.at[:, :, pl.ds(cbase(gi), _B)], sov[b])

        def wait_out(gi, b):
            pltpu.make_async_copy(sobuf[b], os_hbm.at[:, pl.ds(cbase(gi), _B)], sos[b]).wait()
            pltpu.make_async_copy(vobuf[b], ov_hbm.at[:, :, pl.ds(cbase(gi), _B)], sov[b]).wait()

        def work(b):
            @plsc.parallel_loop(0, _B // _L)
            def _tr(gr):
                ev = gr * _L + iota
                for q in range(4):
                    cols = [q * _L + ((iota + kk) & 15) for kk in range(16)]
                    xls = [plsc.load_gather(lbuf[b], [ev, cols[kk]])
                           for kk in range(16)]
                    for kk in range(16):
                        plsc.store_scatter(lbufT, [cols[kk], ev], xls[kk])
                    xrs = [plsc.load_gather(rbuf[b], [ev, cols[kk]])
                           for kk in range(16)]
                    for kk in range(16):
                        plsc.store_scatter(rbufT, [cols[kk], ev], xrs[kk])

            @plsc.parallel_loop(0, _B // _L)
            def _grp(gr):
                sl = pl.ds(gr * _L, _L)
                for c in range(16):
                    ls = lbufT[c, sl]
                    lx = lbufT[16 + c, sl]
                    ly = lbufT[32 + c, sl]
                    lz = lbufT[48 + c, sl]
                    rs = rbufT[c, sl]
                    rx = rbufT[16 + c, sl]
                    ry = rbufT[32 + c, sl]
                    rz = rbufT[48 + c, sl]
                    sobuf[b][c, sl] = ls * rs
                    sobuf[b][16 + c, sl] = lx * rx + ly * ry + lz * rz
                    vobuf[b][0, c, sl] = (lx + rx) * half
                    vobuf[b][1, c, sl] = (ly + ry) * half
                    vobuf[b][2, c, sl] = (lz + rz) * half
                    vobuf[b][0, 16 + c, sl] = ly * rz - lz * ry
                    vobuf[b][1, 16 + c, sl] = lz * rx - lx * rz
                    vobuf[b][2, 16 + c, sl] = lx * ry - ly * rx

        issue_idx(0, 0)

        @pl.when(gw > 1)
        def _():
            issue_idx(1, 1)
        wait_idx(0, 0)
        issue_gather(0)

        @pl.loop(0, (gw + 1) // 2)
        def _pair(g2):
            for b in (0, 1):
                gi = g2 * 2 + b

                @pl.when(gi < gw)
                def _():
                    @pl.when(gi + 1 < gw)
                    def _():
                        wait_idx(gi + 1, 1 - b)
                        issue_gather(1 - b)
                    wait_gather(b)

                    @pl.when(gi + 2 < gw)
                    def _():
                        issue_idx(gi + 2, b)

                    @pl.when(gi >= 2)
                    def _():
                        wait_out(gi - 2, b)
                    work(b)
                    issue_out(gi, b)

        for b in (0, 1):
            @pl.when((gw >= 2) & ((gw - 2) % 2 == b))
            def _():
                wait_out(gw - 2, b)

            @pl.when((gw >= 1) & ((gw - 1) % 2 == b))
            def _():
                wait_out(gw - 1, b)

    return k(lt, rt, li, ri)


def kernel(left_scalar, left_vector, right_scalar, right_vector, left_index, right_index):
    N, S = left_scalar.shape
    V = left_vector.shape[1]
    E = left_index.shape[0]
    pad = jnp.zeros((N, 128 - S - 3 * V), jnp.float32)
    lt = jnp.concatenate(
        [left_scalar, jnp.swapaxes(left_vector, 1, 2).reshape(N, 3 * V), pad], axis=1)
    rt = jnp.concatenate(
        [right_scalar, jnp.swapaxes(right_vector, 1, 2).reshape(N, 3 * V), pad], axis=1)
    so, vo = _fuser_sc(lt, rt, left_index, right_index, E)
    return (so.T, jnp.transpose(vo, (2, 1, 0)))
